# baseline (speedup 1.0000x reference)
import numpy as np
import jax
import jax.numpy as jnp
from jax import lax
from jax.experimental import pallas as pl
from jax.experimental.pallas import tpu as pltpu
from jax.experimental.pallas import tpu_sc as plsc

N = 26
D = 16
BATCH = 4096
NPAIR = N * (N + 1) // 2
NC, NS = 2, 16
NW = NC * NS
BCOL = BATCH // NW
LG = BCOL // 16

IT = 9
NSC = sum(N - i for i in range(IT, N))
NSCP = 160
_KB = {}
_k = 0
for _i in range(IT, N):
    _KB[_i] = _k
    _k += N - _i
BCT = 1024

_TC_ROWS = np.asarray(
    [i * 32 + j for i in range(IT) for j in range(i, N)], np.int32)


def _body(inp, out, xb0, xb1, yb0, yb1, out_v, sems):
    c = lax.axis_index("c")
    s = lax.axis_index("s")
    b0 = (s * NC + c) * BCOL

    xbufs = (xb0, xb1)
    ybufs = (yb0, yb1)

    def slabs(i, h):
        r = N - i
        src_x = inp.at[i, pl.ds(i, r), pl.ds(h * 8, 8), pl.ds(b0, BCOL)]
        src_y = inp.at[pl.ds(i, r), i, pl.ds(h * 8, 8), pl.ds(b0, BCOL)]
        return src_x, src_y

    def start(i, h):
        src_x, src_y = slabs(i, h)
        r = N - i
        pltpu.async_copy(src_x, xbufs[h].at[pl.ds(0, r)], sems[h])
        pltpu.async_copy(src_y, ybufs[h].at[pl.ds(0, r)], sems[h])

    def wait(i, h):
        src_x, src_y = slabs(i, h)
        r = N - i
        pltpu.make_async_copy(src_x, xbufs[h].at[pl.ds(0, r)], sems[h]).wait()
        pltpu.make_async_copy(src_y, ybufs[h].at[pl.ds(0, r)], sems[h]).wait()

    start(IT, 0)
    start(IT, 1)

    for i in range(IT, N):
        r = N - i
        for h in range(2):
            wait(i, h)
            xb = xbufs[h]
            yb = ybufs[h]

            if h == 0:
                @pl.loop(0, r)
                def _pairs0(j):
                    @pl.loop(0, LG, unroll=2)
                    def _lanes0(lg):
                        bs = lg * 16
                        v = xb[j, 0, pl.ds(bs, 16)] * yb[j, 0, pl.ds(bs, 16)]
                        for dd in range(1, 8):
                            v += xb[j, dd, pl.ds(bs, 16)] * yb[j, dd, pl.ds(bs, 16)]
                        out_v[_KB[i] + j, pl.ds(bs, 16)] = v
            else:
                @pl.loop(0, r)
                def _pairs1(j):
                    @pl.loop(0, LG, unroll=2)
                    def _lanes1(lg):
                        bs = lg * 16
                        v = out_v[_KB[i] + j, pl.ds(bs, 16)]
                        for dd in range(8):
                            v += xb[j, dd, pl.ds(bs, 16)] * yb[j, dd, pl.ds(bs, 16)]
                        out_v[_KB[i] + j, pl.ds(bs, 16)] = v

            if i + 1 < N:
                start(i + 1, h)

    pltpu.sync_copy(out_v, out.at[pl.ds(0, NSCP), pl.ds(b0, BCOL)])


def _tc_body(x_ref, y_ref, o_ref):
    o_ref[0, pl.ds(0, N)] = jnp.sum(x_ref[0] * y_ref[:, 0], axis=1)


@jax.jit
def kernel(input):
    inp = jnp.transpose(input, (1, 2, 0)).reshape(N, N, D, BATCH)
    mesh = plsc.VectorSubcoreMesh(
        core_axis_name="c", subcore_axis_name="s",
        num_cores=NC, num_subcores=NS)
    f = pl.kernel(
        _body,
        out_type=jax.ShapeDtypeStruct((NSCP, BATCH), jnp.float32),
        mesh=mesh,
        compiler_params=pltpu.CompilerParams(needs_layout_passes=False),
        scratch_types=[
            pltpu.VMEM((N - IT, 8, BCOL), jnp.float32),
            pltpu.VMEM((N - IT, 8, BCOL), jnp.float32),
            pltpu.VMEM((N - IT, 8, BCOL), jnp.float32),
            pltpu.VMEM((N - IT, 8, BCOL), jnp.float32),
            pltpu.VMEM((NSCP, BCOL), jnp.float32),
            (pltpu.SemaphoreType.DMA, pltpu.SemaphoreType.DMA),
        ],
    )
    sc_out = f(inp)
    tc = pl.pallas_call(
        _tc_body,
        grid=(IT, BATCH // BCT),
        in_specs=[
            pl.BlockSpec((1, N, D, BCT), lambda ib, cb: (ib, 0, 0, cb)),
            pl.BlockSpec((N, 1, D, BCT), lambda ib, cb: (0, ib, 0, cb)),
        ],
        out_specs=pl.BlockSpec((1, 32, BCT), lambda ib, cb: (ib, 0, cb)),
        out_shape=jax.ShapeDtypeStruct((IT, 32, BATCH), jnp.float32),
    )
    tc_out = tc(inp, inp)
    a = jnp.take(tc_out.reshape(IT * 32, BATCH), jnp.asarray(_TC_ROWS), axis=0,
                 mode="clip")
    final = jnp.zeros((NPAIR, BATCH), jnp.float32)
    final = lax.dynamic_update_slice(final, a, (0, 0))
    final = lax.dynamic_update_slice(final, sc_out[:NSC], (NPAIR - NSC, 0))
    return final.T


# speedup vs baseline: 1.0618x; 1.0618x over previous
import numpy as np
import jax
import jax.numpy as jnp
from jax import lax
from jax.experimental import pallas as pl
from jax.experimental.pallas import tpu as pltpu
from jax.experimental.pallas import tpu_sc as plsc

N = 26
D = 16
BATCH = 4096
NPAIR = N * (N + 1) // 2
NC, NS = 2, 16
NW = NC * NS
BCOL = BATCH // NW
LG = BCOL // 16

IT = 9
NSC = sum(N - i for i in range(IT, N))
NSCP = 160
_KB = {}
_k = 0
for _i in range(IT, N):
    _KB[_i] = _k
    _k += N - _i
BCT = 1024

_TC_ROWS = np.asarray(
    [i * 32 + j for i in range(IT) for j in range(i, N)], np.int32)


def _body(inp, out, xb0, xb1, yb0, yb1, out_v, sems):
    c = lax.axis_index("c")
    s = lax.axis_index("s")
    b0 = (s * NC + c) * BCOL

    xbufs = (xb0, xb1)
    ybufs = (yb0, yb1)

    def slabs(i, h):
        r = N - i
        src_x = inp.at[i, pl.ds(i, r), pl.ds(h * 8, 8), pl.ds(b0, BCOL)]
        src_y = inp.at[pl.ds(i, r), i, pl.ds(h * 8, 8), pl.ds(b0, BCOL)]
        return src_x, src_y

    def start(i, h):
        src_x, src_y = slabs(i, h)
        r = N - i
        pltpu.async_copy(src_x, xbufs[h].at[pl.ds(0, r)], sems[h])
        pltpu.async_copy(src_y, ybufs[h].at[pl.ds(0, r)], sems[h])

    def wait(i, h):
        src_x, src_y = slabs(i, h)
        r = N - i
        pltpu.make_async_copy(src_x, xbufs[h].at[pl.ds(0, r)], sems[h]).wait()
        pltpu.make_async_copy(src_y, ybufs[h].at[pl.ds(0, r)], sems[h]).wait()

    start(IT, 0)
    start(IT, 1)

    for i in range(IT, N):
        r = N - i
        for h in range(2):
            wait(i, h)
            xb = xbufs[h]
            yb = ybufs[h]

            if h == 0:
                @pl.loop(0, r)
                def _pairs0(j):
                    @pl.loop(0, LG, unroll=2)
                    def _lanes0(lg):
                        bs = lg * 16
                        v = xb[j, 0, pl.ds(bs, 16)] * yb[j, 0, pl.ds(bs, 16)]
                        for dd in range(1, 8):
                            v += xb[j, dd, pl.ds(bs, 16)] * yb[j, dd, pl.ds(bs, 16)]
                        out_v[_KB[i] + j, pl.ds(bs, 16)] = v
            else:
                @pl.loop(0, r)
                def _pairs1(j):
                    @pl.loop(0, LG, unroll=2)
                    def _lanes1(lg):
                        bs = lg * 16
                        v = out_v[_KB[i] + j, pl.ds(bs, 16)]
                        for dd in range(8):
                            v += xb[j, dd, pl.ds(bs, 16)] * yb[j, dd, pl.ds(bs, 16)]
                        out_v[_KB[i] + j, pl.ds(bs, 16)] = v

            if i + 1 < N:
                start(i + 1, h)

    pltpu.sync_copy(out_v, out.at[pl.ds(0, NSCP), pl.ds(b0, BCOL)])


def _tc_body(x_ref, y_ref, o_ref):
    o_ref[0, pl.ds(0, N)] = jnp.sum(x_ref[0] * y_ref[:, 0], axis=1)


@jax.jit
def kernel(input):
    inp = jnp.transpose(input, (1, 2, 0)).reshape(N, N, D, BATCH)
    mesh = plsc.VectorSubcoreMesh(
        core_axis_name="c", subcore_axis_name="s",
        num_cores=NC, num_subcores=NS)
    f = pl.kernel(
        _body,
        out_type=jax.ShapeDtypeStruct((NSCP, BATCH), jnp.float32),
        mesh=mesh,
        compiler_params=pltpu.CompilerParams(needs_layout_passes=False),
        scratch_types=[
            pltpu.VMEM((N - IT, 8, BCOL), jnp.float32),
            pltpu.VMEM((N - IT, 8, BCOL), jnp.float32),
            pltpu.VMEM((N - IT, 8, BCOL), jnp.float32),
            pltpu.VMEM((N - IT, 8, BCOL), jnp.float32),
            pltpu.VMEM((NSCP, BCOL), jnp.float32),
            (pltpu.SemaphoreType.DMA, pltpu.SemaphoreType.DMA),
        ],
    )
    sc_out = f(inp)
    tc = pl.pallas_call(
        _tc_body,
        grid=(IT, BATCH // BCT),
        in_specs=[
            pl.BlockSpec((1, N, D, BCT), lambda ib, cb: (ib, 0, 0, cb)),
            pl.BlockSpec((N, 1, D, BCT), lambda ib, cb: (0, ib, 0, cb)),
        ],
        out_specs=pl.BlockSpec((1, 32, BCT), lambda ib, cb: (ib, 0, cb)),
        out_shape=jax.ShapeDtypeStruct((IT, 32, BATCH), jnp.float32),
    )
    tc_out = tc(inp, inp)
    a = jnp.take(tc_out.reshape(IT * 32, BATCH), jnp.asarray(_TC_ROWS), axis=0,
                 mode="clip")
    return jnp.concatenate([a, sc_out[:NSC]], axis=0).T
